# Initial kernel scaffold; baseline (speedup 1.0000x reference)
import jax
import jax.numpy as jnp
from jax.experimental import pallas as pl


def kernel(x, edge_index, batch, W1, b1, W2, b2, Wc0, bc0, Wc1, bc1, Wf1, bf1, Wf2, bf2, Wf3, bf3):
    raise NotImplementedError("write your pallas kernel here")


# speedup vs baseline: 2.2612x; 2.2612x over previous
import functools

import jax
import jax.numpy as jnp
from jax.experimental import pallas as pl
from jax.experimental.pallas import tpu as pltpu

N = 10000
N_GRAPHS = 64


def _mm_body(x_ref, w_ref, o_ref):
    o_ref[...] = jnp.dot(x_ref[...], w_ref[...],
                         preferred_element_type=jnp.float32)


def _pallas_matmul(x, w):
    m, k = x.shape
    _, n = w.shape
    bm = 512
    grid = (pl.cdiv(m, bm),)
    return pl.pallas_call(
        _mm_body,
        grid=grid,
        in_specs=[
            pl.BlockSpec((bm, k), lambda i: (i, 0)),
            pl.BlockSpec((k, n), lambda i: (0, 0)),
        ],
        out_specs=pl.BlockSpec((bm, n), lambda i: (i, 0)),
        out_shape=jax.ShapeDtypeStruct((m, n), jnp.float32),
    )(x, w)


def _conv(h, b, src, dst, dinv):
    g = dinv[:, None] * h
    agg = jnp.zeros_like(g).at[dst].add(jnp.take(g, src, axis=0))
    return dinv[:, None] * (agg + g) + b


def kernel(x, edge_index, batch, W1, b1, W2, b2, Wc0, bc0, Wc1, bc1,
           Wf1, bf1, Wf2, bf2, Wf3, bf3):
    src, dst = edge_index[0], edge_index[1]
    act = jax.nn.elu
    deg = jnp.zeros((N,), jnp.float32).at[dst].add(1.0) + 1.0
    dinv = jax.lax.rsqrt(deg)

    h1 = _pallas_matmul(x, W1)
    x1 = act(_conv(h1, b1, src, dst, dinv))
    x3 = act(_conv(_pallas_matmul(x1, W2), b2, src, dst, dinv))
    x3 = act(_conv(_pallas_matmul(x3, Wc0), bc0, src, dst, dinv))
    x3 = act(_conv(_pallas_matmul(x3, Wc1), bc1, src, dst, dinv))

    pooled = jax.ops.segment_max(x3, batch, num_segments=N_GRAPHS,
                                 indices_are_sorted=True)
    pooled = jnp.where(jnp.isfinite(pooled), pooled, 0.0)
    h = act(pooled @ Wf1 + bf1)
    h = act(h @ Wf2 + bf2)
    logits = h @ Wf3 + bf3
    return jax.nn.log_softmax(logits, axis=1)


# speedup vs baseline: 13.8387x; 6.1200x over previous
import functools

import jax
import jax.numpy as jnp
from jax import lax
from jax.experimental import pallas as pl
from jax.experimental.pallas import tpu as pltpu
from jax.experimental.pallas import tpu_sc as plsc

N = 10000
E = 320000
N_GRAPHS = 64

NC = 2
NS = 16
NW = NC * NS
CHUNK = 128
NCHUNK = (E + NW * CHUNK - 1) // (NW * CHUNK)
E_PAD = NW * NCHUNK * CHUNK
STRIPE = 640
NP = NS * STRIPE


def _make_sc_agg(F):
    mesh = plsc.VectorSubcoreMesh(core_axis_name="c", subcore_axis_name="s")

    @functools.partial(
        pl.kernel, mesh=mesh,
        compiler_params=pltpu.CompilerParams(use_tc_tiling_on_sc=False),
        out_type=jax.ShapeDtypeStruct((NW, STRIPE, F), jnp.float32),
        scratch_types=[
            pltpu.VMEM((NCHUNK, CHUNK), jnp.int32),
            pltpu.VMEM((NCHUNK, CHUNK), jnp.int32),
            pltpu.VMEM((CHUNK, F), jnp.float32),
            pltpu.VMEM((64, F), jnp.float32),
            pltpu.VMEM_SHARED((NP, F), jnp.float32),
            pltpu.SemaphoreType.DMA,
        ],
    )
    def k(g_hbm, srcR, dstR, out_hbm, src_v, dst_v, buf, zbuf, acc, sem):
        c = lax.axis_index("c")
        s = lax.axis_index("s")
        wid = c * NS + s

        for i in range(64):
            for j in range(F // 16):
                zbuf[i, 16 * j:16 * (j + 1)] = jnp.zeros((16,), jnp.float32)

        def zstripe(kk, carry):
            pltpu.sync_copy(zbuf, acc.at[pl.ds(s * STRIPE + kk * 64, 64)])
            return carry
        lax.fori_loop(0, STRIPE // 64, zstripe, 0)
        plsc.subcore_barrier()

        pltpu.sync_copy(srcR.at[wid], src_v)
        pltpu.sync_copy(dstR.at[wid], dst_v)

        def chunk(j, carry):
            pltpu.async_copy(g_hbm.at[src_v.at[j]], buf, sem).wait()
            pltpu.sync_copy(buf, acc.at[dst_v.at[j]], add=True)
            return carry
        lax.fori_loop(0, NCHUNK, chunk, 0)

        plsc.subcore_barrier()
        pltpu.sync_copy(acc.at[pl.ds(s * STRIPE, STRIPE)], out_hbm.at[wid])

    return k


_sc_agg = {f: _make_sc_agg(f) for f in (16, 32, 64)}


def _mm_body(x_ref, w_ref, o_ref):
    o_ref[...] = jnp.dot(x_ref[...], w_ref[...],
                         preferred_element_type=jnp.float32)


def _pallas_matmul(x, w):
    m, k = x.shape
    _, n = w.shape
    bm = 512
    grid = (pl.cdiv(m, bm),)
    return pl.pallas_call(
        _mm_body,
        grid=grid,
        in_specs=[
            pl.BlockSpec((bm, k), lambda i: (i, 0)),
            pl.BlockSpec((k, n), lambda i: (0, 0)),
        ],
        out_specs=pl.BlockSpec((bm, n), lambda i: (i, 0)),
        out_shape=jax.ShapeDtypeStruct((m, n), jnp.float32),
    )(x, w)


def _agg(g, srcR, dstR):
    p = _sc_agg[g.shape[1]](g, srcR, dstR).reshape(NC, NP, g.shape[1])
    return p[0, :N] + p[1, :N]


def kernel(x, edge_index, batch, W1, b1, W2, b2, Wc0, bc0, Wc1, bc1,
           Wf1, bf1, Wf2, bf2, Wf3, bf3):
    src, dst = edge_index[0], edge_index[1]
    act = jax.nn.elu

    pad = E_PAD - E
    srcR = jnp.concatenate([src, jnp.zeros((pad,), jnp.int32)]) \
        .reshape(NW, NCHUNK, CHUNK)
    dstR = jnp.concatenate([dst, jnp.full((pad,), N, jnp.int32)]) \
        .reshape(NW, NCHUNK, CHUNK)

    ones16 = jnp.ones((N, 16), jnp.float32)
    deg = _agg(ones16, srcR, dstR)[:, 0] + 1.0
    dinv = jax.lax.rsqrt(deg)

    def conv(h, b):
        g = dinv[:, None] * h
        return dinv[:, None] * (_agg(g, srcR, dstR) + g) + b

    x1 = act(conv(_pallas_matmul(x, W1), b1))
    x3 = act(conv(_pallas_matmul(x1, W2), b2))
    x3 = act(conv(_pallas_matmul(x3, Wc0), bc0))
    x3 = act(conv(_pallas_matmul(x3, Wc1), bc1))

    pooled = jax.ops.segment_max(x3, batch, num_segments=N_GRAPHS,
                                 indices_are_sorted=True)
    pooled = jnp.where(jnp.isfinite(pooled), pooled, 0.0)
    h = act(pooled @ Wf1 + bf1)
    h = act(h @ Wf2 + bf2)
    logits = h @ Wf3 + bf3
    return jax.nn.log_softmax(logits, axis=1)
